# baseline (speedup 1.0000x reference)
import jax
import jax.numpy as jnp
from jax import lax
from jax.experimental import pallas as pl
from jax.experimental.pallas import tpu as pltpu
from jax.experimental.pallas import tpu_sc as plsc

_B = 128
_TAB = 1000
_LANES = 16


def _sc_gather_body(t_hbm, sac_hbm, som_hbm, a_hbm, s_hbm,
                    t_v, sac_v, som_v, a_v, s_v):
    cid = lax.axis_index("c")
    sid = lax.axis_index("s")

    @pl.when(jnp.logical_and(cid == 0, sid == 0))
    def _():
        pltpu.sync_copy(t_hbm, t_v)
        pltpu.sync_copy(sac_hbm, sac_v)
        pltpu.sync_copy(som_hbm, som_v)
        for i in range(_B // _LANES):
            idx = t_v[pl.ds(i * _LANES, _LANES)]
            a_v[pl.ds(i * _LANES, _LANES)] = plsc.load_gather(sac_v, [idx])
            s_v[pl.ds(i * _LANES, _LANES)] = plsc.load_gather(som_v, [idx])
        pltpu.sync_copy(a_v, a_hbm)
        pltpu.sync_copy(s_v, s_hbm)


_SC_GATHER_CACHE = []


def _sc_gather():
    if not _SC_GATHER_CACHE:
        _SC_GATHER_CACHE.append(pl.kernel(
            _sc_gather_body,
            out_type=(jax.ShapeDtypeStruct((_B,), jnp.float32),
                      jax.ShapeDtypeStruct((_B,), jnp.float32)),
            mesh=plsc.VectorSubcoreMesh(core_axis_name="c",
                                        subcore_axis_name="s"),
            compiler_params=pltpu.CompilerParams(needs_layout_passes=False),
            scratch_types=[
                pltpu.VMEM((_B,), jnp.int32),
                pltpu.VMEM((_TAB,), jnp.float32),
                pltpu.VMEM((_TAB,), jnp.float32),
                pltpu.VMEM((_B,), jnp.float32),
                pltpu.VMEM((_B,), jnp.float32),
            ],
        ))
    return _SC_GATHER_CACHE[0]


_BB = 1


def _tc_fma_body(a_sref, s_sref, x_ref, n_ref, o_ref):
    i = pl.program_id(0)
    o_ref[...] = a_sref[i] * x_ref[...] + s_sref[i] * n_ref[...]


def _tc_fma(a, s, x4, n4):
    grid_spec = pltpu.PrefetchScalarGridSpec(
        num_scalar_prefetch=2,
        grid=(_B // _BB,),
        in_specs=[
            pl.BlockSpec((_BB, 3, 64, 64), lambda i, a_s, s_s: (i, 0, 0, 0)),
            pl.BlockSpec((_BB, 3, 64, 64), lambda i, a_s, s_s: (i, 0, 0, 0)),
        ],
        out_specs=pl.BlockSpec((_BB, 3, 64, 64), lambda i, a_s, s_s: (i, 0, 0, 0)),
    )
    return pl.pallas_call(
        _tc_fma_body,
        grid_spec=grid_spec,
        out_shape=jax.ShapeDtypeStruct((_B, 3, 64, 64), jnp.float32),
    )(a, s, x4, n4)


def kernel(x0, t, noise, sqrt_alphas_cumprod, sqrt_one_minus_alphas_cumprod):
    a, s = _sc_gather()(t.astype(jnp.int32), sqrt_alphas_cumprod,
                        sqrt_one_minus_alphas_cumprod)
    x_t = _tc_fma(a, s, x0, noise)
    return (x_t, noise)


# speedup vs baseline: 1.1329x; 1.1329x over previous
import jax
import jax.numpy as jnp
from jax import lax
from jax.experimental import pallas as pl
from jax.experimental.pallas import tpu as pltpu
from jax.experimental.pallas import tpu_sc as plsc

_B = 128
_TAB = 1000
_LANES = 16


def _sc_gather_body(t_hbm, sac_hbm, som_hbm, a_hbm, s_hbm,
                    t_v, sac_v, som_v, a_v, s_v):
    cid = lax.axis_index("c")
    sid = lax.axis_index("s")

    @pl.when(jnp.logical_and(cid == 0, sid == 0))
    def _():
        pltpu.sync_copy(t_hbm, t_v)
        pltpu.sync_copy(sac_hbm, sac_v)
        pltpu.sync_copy(som_hbm, som_v)
        for i in range(_B // _LANES):
            idx = t_v[pl.ds(i * _LANES, _LANES)]
            a_v[pl.ds(i * _LANES, _LANES)] = plsc.load_gather(sac_v, [idx])
            s_v[pl.ds(i * _LANES, _LANES)] = plsc.load_gather(som_v, [idx])
        pltpu.sync_copy(a_v, a_hbm)
        pltpu.sync_copy(s_v, s_hbm)


_SC_GATHER_CACHE = []


def _sc_gather():
    if not _SC_GATHER_CACHE:
        _SC_GATHER_CACHE.append(pl.kernel(
            _sc_gather_body,
            out_type=(jax.ShapeDtypeStruct((_B,), jnp.float32),
                      jax.ShapeDtypeStruct((_B,), jnp.float32)),
            mesh=plsc.VectorSubcoreMesh(core_axis_name="c",
                                        subcore_axis_name="s"),
            compiler_params=pltpu.CompilerParams(needs_layout_passes=False),
            scratch_types=[
                pltpu.VMEM((_B,), jnp.int32),
                pltpu.VMEM((_TAB,), jnp.float32),
                pltpu.VMEM((_TAB,), jnp.float32),
                pltpu.VMEM((_B,), jnp.float32),
                pltpu.VMEM((_B,), jnp.float32),
            ],
        ))
    return _SC_GATHER_CACHE[0]


_BB = 1


def _tc_fma_body(a_sref, s_sref, x_ref, n_ref, o_ref):
    i = pl.program_id(0)
    o_ref[...] = a_sref[i] * x_ref[...] + s_sref[i] * n_ref[...]


def _tc_fma(a, s, x4, n4):
    grid_spec = pltpu.PrefetchScalarGridSpec(
        num_scalar_prefetch=2,
        grid=(_B // _BB,),
        in_specs=[
            pl.BlockSpec((_BB, 3, 64, 64), lambda i, a_s, s_s: (i, 0, 0, 0)),
            pl.BlockSpec((_BB, 3, 64, 64), lambda i, a_s, s_s: (i, 0, 0, 0)),
        ],
        out_specs=pl.BlockSpec((_BB, 3, 64, 64), lambda i, a_s, s_s: (i, 0, 0, 0)),
    )
    return pl.pallas_call(
        _tc_fma_body,
        grid_spec=grid_spec,
        out_shape=jax.ShapeDtypeStruct((_B, 3, 64, 64), jnp.float32),
    )(a, s, x4, n4)


def _tc_all_body(t_sref, sac_sref, som_sref, x_ref, n_ref, o_ref):
    i = pl.program_id(0)
    ti = t_sref[i]
    o_ref[...] = sac_sref[ti] * x_ref[...] + som_sref[ti] * n_ref[...]


def _tc_all(t, sac, som, x4, n4):
    grid_spec = pltpu.PrefetchScalarGridSpec(
        num_scalar_prefetch=3,
        grid=(_B // _BB,),
        in_specs=[
            pl.BlockSpec((_BB, 3, 64, 64), lambda i, *_: (i, 0, 0, 0)),
            pl.BlockSpec((_BB, 3, 64, 64), lambda i, *_: (i, 0, 0, 0)),
        ],
        out_specs=pl.BlockSpec((_BB, 3, 64, 64), lambda i, *_: (i, 0, 0, 0)),
    )
    return pl.pallas_call(
        _tc_all_body,
        grid_spec=grid_spec,
        out_shape=jax.ShapeDtypeStruct((_B, 3, 64, 64), jnp.float32),
    )(t, sac, som, x4, n4)


def kernel(x0, t, noise, sqrt_alphas_cumprod, sqrt_one_minus_alphas_cumprod):
    x_t = _tc_all(t.astype(jnp.int32), sqrt_alphas_cumprod,
                  sqrt_one_minus_alphas_cumprod, x0, noise)
    return (x_t, noise)


# speedup vs baseline: 2.0997x; 1.8535x over previous
import jax
import jax.numpy as jnp
from jax import lax
from jax.experimental import pallas as pl
from jax.experimental.pallas import tpu as pltpu
from jax.experimental.pallas import tpu_sc as plsc

_B = 128
_TAB = 1000
_LANES = 16


def _sc_gather_body(t_hbm, sac_hbm, som_hbm, a_hbm, s_hbm,
                    t_v, sac_v, som_v, a_v, s_v):
    cid = lax.axis_index("c")
    sid = lax.axis_index("s")

    @pl.when(jnp.logical_and(cid == 0, sid == 0))
    def _():
        pltpu.sync_copy(t_hbm, t_v)
        pltpu.sync_copy(sac_hbm, sac_v)
        pltpu.sync_copy(som_hbm, som_v)
        for i in range(_B // _LANES):
            idx = t_v[pl.ds(i * _LANES, _LANES)]
            a_v[pl.ds(i * _LANES, _LANES)] = plsc.load_gather(sac_v, [idx])
            s_v[pl.ds(i * _LANES, _LANES)] = plsc.load_gather(som_v, [idx])
        pltpu.sync_copy(a_v, a_hbm)
        pltpu.sync_copy(s_v, s_hbm)


_SC_GATHER_CACHE = []


def _sc_gather():
    if not _SC_GATHER_CACHE:
        _SC_GATHER_CACHE.append(pl.kernel(
            _sc_gather_body,
            out_type=(jax.ShapeDtypeStruct((_B,), jnp.float32),
                      jax.ShapeDtypeStruct((_B,), jnp.float32)),
            mesh=plsc.VectorSubcoreMesh(core_axis_name="c",
                                        subcore_axis_name="s"),
            compiler_params=pltpu.CompilerParams(needs_layout_passes=False),
            scratch_types=[
                pltpu.VMEM((_B,), jnp.int32),
                pltpu.VMEM((_TAB,), jnp.float32),
                pltpu.VMEM((_TAB,), jnp.float32),
                pltpu.VMEM((_B,), jnp.float32),
                pltpu.VMEM((_B,), jnp.float32),
            ],
        ))
    return _SC_GATHER_CACHE[0]


_BB = 8


def _tc_fma_body(a_sref, s_sref, x_ref, n_ref, o_ref):
    i = pl.program_id(0)
    o_ref[...] = a_sref[i] * x_ref[...] + s_sref[i] * n_ref[...]


def _tc_fma(a, s, x4, n4):
    grid_spec = pltpu.PrefetchScalarGridSpec(
        num_scalar_prefetch=2,
        grid=(_B // _BB,),
        in_specs=[
            pl.BlockSpec((_BB, 3, 64, 64), lambda i, a_s, s_s: (i, 0, 0, 0)),
            pl.BlockSpec((_BB, 3, 64, 64), lambda i, a_s, s_s: (i, 0, 0, 0)),
        ],
        out_specs=pl.BlockSpec((_BB, 3, 64, 64), lambda i, a_s, s_s: (i, 0, 0, 0)),
    )
    return pl.pallas_call(
        _tc_fma_body,
        grid_spec=grid_spec,
        out_shape=jax.ShapeDtypeStruct((_B, 3, 64, 64), jnp.float32),
    )(a, s, x4, n4)


def _tc_all_body(t_sref, sac_sref, som_sref, x_ref, n_ref, o_ref):
    i = pl.program_id(0)
    for j in range(_BB):
        ti = t_sref[i * _BB + j]
        o_ref[j] = sac_sref[ti] * x_ref[j] + som_sref[ti] * n_ref[j]


def _tc_all(t, sac, som, x4, n4):
    grid_spec = pltpu.PrefetchScalarGridSpec(
        num_scalar_prefetch=3,
        grid=(_B // _BB,),
        in_specs=[
            pl.BlockSpec((_BB, 3, 64, 64), lambda i, *_: (i, 0, 0, 0)),
            pl.BlockSpec((_BB, 3, 64, 64), lambda i, *_: (i, 0, 0, 0)),
        ],
        out_specs=pl.BlockSpec((_BB, 3, 64, 64), lambda i, *_: (i, 0, 0, 0)),
    )
    return pl.pallas_call(
        _tc_all_body,
        grid_spec=grid_spec,
        out_shape=jax.ShapeDtypeStruct((_B, 3, 64, 64), jnp.float32),
    )(t, sac, som, x4, n4)


def kernel(x0, t, noise, sqrt_alphas_cumprod, sqrt_one_minus_alphas_cumprod):
    x_t = _tc_all(t.astype(jnp.int32), sqrt_alphas_cumprod,
                  sqrt_one_minus_alphas_cumprod, x0, noise)
    return (x_t, noise)


# speedup vs baseline: 2.2226x; 1.0585x over previous
import jax
import jax.numpy as jnp
from jax import lax
from jax.experimental import pallas as pl
from jax.experimental.pallas import tpu as pltpu
from jax.experimental.pallas import tpu_sc as plsc

_B = 128
_TAB = 1000
_LANES = 16


def _sc_gather_body(t_hbm, sac_hbm, som_hbm, a_hbm, s_hbm,
                    t_v, sac_v, som_v, a_v, s_v):
    cid = lax.axis_index("c")
    sid = lax.axis_index("s")

    @pl.when(jnp.logical_and(cid == 0, sid == 0))
    def _():
        pltpu.sync_copy(t_hbm, t_v)
        pltpu.sync_copy(sac_hbm, sac_v)
        pltpu.sync_copy(som_hbm, som_v)
        for i in range(_B // _LANES):
            idx = t_v[pl.ds(i * _LANES, _LANES)]
            a_v[pl.ds(i * _LANES, _LANES)] = plsc.load_gather(sac_v, [idx])
            s_v[pl.ds(i * _LANES, _LANES)] = plsc.load_gather(som_v, [idx])
        pltpu.sync_copy(a_v, a_hbm)
        pltpu.sync_copy(s_v, s_hbm)


_SC_GATHER_CACHE = []


def _sc_gather():
    if not _SC_GATHER_CACHE:
        _SC_GATHER_CACHE.append(pl.kernel(
            _sc_gather_body,
            out_type=(jax.ShapeDtypeStruct((_B,), jnp.float32),
                      jax.ShapeDtypeStruct((_B,), jnp.float32)),
            mesh=plsc.VectorSubcoreMesh(core_axis_name="c",
                                        subcore_axis_name="s"),
            compiler_params=pltpu.CompilerParams(needs_layout_passes=False),
            scratch_types=[
                pltpu.VMEM((_B,), jnp.int32),
                pltpu.VMEM((_TAB,), jnp.float32),
                pltpu.VMEM((_TAB,), jnp.float32),
                pltpu.VMEM((_B,), jnp.float32),
                pltpu.VMEM((_B,), jnp.float32),
            ],
        ))
    return _SC_GATHER_CACHE[0]


_BB = 16


def _tc_fma_body(a_sref, s_sref, x_ref, n_ref, o_ref):
    i = pl.program_id(0)
    o_ref[...] = a_sref[i] * x_ref[...] + s_sref[i] * n_ref[...]


def _tc_fma(a, s, x4, n4):
    grid_spec = pltpu.PrefetchScalarGridSpec(
        num_scalar_prefetch=2,
        grid=(_B // _BB,),
        in_specs=[
            pl.BlockSpec((_BB, 3, 64, 64), lambda i, a_s, s_s: (i, 0, 0, 0)),
            pl.BlockSpec((_BB, 3, 64, 64), lambda i, a_s, s_s: (i, 0, 0, 0)),
        ],
        out_specs=pl.BlockSpec((_BB, 3, 64, 64), lambda i, a_s, s_s: (i, 0, 0, 0)),
    )
    return pl.pallas_call(
        _tc_fma_body,
        grid_spec=grid_spec,
        out_shape=jax.ShapeDtypeStruct((_B, 3, 64, 64), jnp.float32),
    )(a, s, x4, n4)


def _tc_all_body(t_sref, sac_sref, som_sref, x_ref, n_ref, o_ref):
    i = pl.program_id(0)
    for j in range(_BB):
        ti = t_sref[i * _BB + j]
        o_ref[j] = sac_sref[ti] * x_ref[j] + som_sref[ti] * n_ref[j]


def _tc_all(t, sac, som, x4, n4):
    grid_spec = pltpu.PrefetchScalarGridSpec(
        num_scalar_prefetch=3,
        grid=(_B // _BB,),
        in_specs=[
            pl.BlockSpec((_BB, 3, 64, 64), lambda i, *_: (i, 0, 0, 0)),
            pl.BlockSpec((_BB, 3, 64, 64), lambda i, *_: (i, 0, 0, 0)),
        ],
        out_specs=pl.BlockSpec((_BB, 3, 64, 64), lambda i, *_: (i, 0, 0, 0)),
    )
    return pl.pallas_call(
        _tc_all_body,
        grid_spec=grid_spec,
        out_shape=jax.ShapeDtypeStruct((_B, 3, 64, 64), jnp.float32),
    )(t, sac, som, x4, n4)


def kernel(x0, t, noise, sqrt_alphas_cumprod, sqrt_one_minus_alphas_cumprod):
    x_t = _tc_all(t.astype(jnp.int32), sqrt_alphas_cumprod,
                  sqrt_one_minus_alphas_cumprod, x0, noise)
    return (x_t, noise)


# speedup vs baseline: 10.1585x; 4.5705x over previous
import jax
import jax.numpy as jnp
from jax import lax
from jax.experimental import pallas as pl
from jax.experimental.pallas import tpu as pltpu
from jax.experimental.pallas import tpu_sc as plsc

_B = 128
_TAB = 1000
_ROWS = 3 * 64 * 64
_G = 8
_BLK = _ROWS // _G


def _tc_body(t_sref, sac_sref, som_sref, x_ref, n_ref, o_ref, no_ref,
             a_scr, s_scr):
    i = pl.program_id(0)

    @pl.when(i == 0)
    def _():
        lane = lax.broadcasted_iota(jnp.int32, (1, _B), 1)
        a_row = jnp.zeros((1, _B), jnp.float32)
        s_row = jnp.zeros((1, _B), jnp.float32)
        for j in range(_B):
            tj = t_sref[j]
            a_row = jnp.where(lane == j, sac_sref[tj], a_row)
            s_row = jnp.where(lane == j, som_sref[tj], s_row)
        a_scr[0:1, :] = a_row
        s_scr[0:1, :] = s_row

    a = a_scr[0:1, :]
    s = s_scr[0:1, :]
    n = n_ref[...]
    o_ref[...] = a * x_ref[...] + s * n
    no_ref[...] = n


def _tc_fma(t, sac, som, x2, n2):
    grid_spec = pltpu.PrefetchScalarGridSpec(
        num_scalar_prefetch=3,
        grid=(_G,),
        in_specs=[
            pl.BlockSpec((_BLK, _B), lambda i, *_: (i, 0)),
            pl.BlockSpec((_BLK, _B), lambda i, *_: (i, 0)),
        ],
        out_specs=[
            pl.BlockSpec((_BLK, _B), lambda i, *_: (i, 0)),
            pl.BlockSpec((_BLK, _B), lambda i, *_: (i, 0)),
        ],
        scratch_shapes=[
            pltpu.VMEM((8, _B), jnp.float32),
            pltpu.VMEM((8, _B), jnp.float32),
        ],
    )
    return pl.pallas_call(
        _tc_body,
        grid_spec=grid_spec,
        out_shape=(jax.ShapeDtypeStruct((_ROWS, _B), jnp.float32),
                   jax.ShapeDtypeStruct((_ROWS, _B), jnp.float32)),
    )(t, sac, som, x2, n2)


def kernel(x0, t, noise, sqrt_alphas_cumprod, sqrt_one_minus_alphas_cumprod):
    x2 = jnp.transpose(x0, (1, 2, 3, 0)).reshape(_ROWS, _B)
    n2 = jnp.transpose(noise, (1, 2, 3, 0)).reshape(_ROWS, _B)
    xt2, no2 = _tc_fma(t.astype(jnp.int32), sqrt_alphas_cumprod,
                       sqrt_one_minus_alphas_cumprod, x2, n2)
    x_t = jnp.transpose(xt2.reshape(3, 64, 64, _B), (3, 0, 1, 2))
    n_out = jnp.transpose(no2.reshape(3, 64, 64, _B), (3, 0, 1, 2))
    return (x_t, n_out)
